# baseline (speedup 1.0000x reference)
import functools

import jax
import jax.numpy as jnp
from jax import lax
from jax.experimental import pallas as pl
from jax.experimental.pallas import tpu as pltpu
from jax.experimental.pallas import tpu_sc as plsc

N_NODES = 10000
D_STATE = 32
D_PK = D_STATE // 2
NC, NS = 2, 16
NW = NC * NS
E_BLK = 128
EDGE_PAD = 327680
PER_W = EDGE_PAD // NW
NBLK = PER_W // E_BLK
EPH = EDGE_PAD // 2
NBLK_H = NBLK // 2
N_PAD = 10016
STRIPE = N_PAD // NS
_GK = 4

_MESH = dict(core_axis_name="c", subcore_axis_name="s")


def _sc_gather(h_pk, src_idx, dst_idx):
    out = (jax.ShapeDtypeStruct((EDGE_PAD, D_PK), jnp.float32),
           jax.ShapeDtypeStruct((EDGE_PAD, D_PK), jnp.float32))

    @functools.partial(
        pl.kernel, mesh=plsc.VectorSubcoreMesh(**_MESH), out_type=out,
        compiler_params=pltpu.CompilerParams(use_tc_tiling_on_sc=False),
        scratch_types=[
            pltpu.VMEM_SHARED((N_NODES, D_PK), jnp.float32),
            pltpu.VMEM((PER_W,), jnp.int32),
            pltpu.VMEM((PER_W,), jnp.int32),
            pltpu.VMEM((_GK, E_BLK, D_PK), jnp.float32),
            pltpu.VMEM((_GK, E_BLK, D_PK), jnp.float32),
            pltpu.SemaphoreType.DMA,
            pltpu.SemaphoreType.DMA,
        ])
    def k(h_hbm, src_hbm, dst_hbm, hs_hbm, hd_hbm, h_sh, idx_s, idx_d, buf_s,
          buf_d, gsem, wsem):
        sid = lax.axis_index("s")
        wid = sid * NC + lax.axis_index("c")
        base = wid * PER_W
        pltpu.sync_copy(h_hbm.at[pl.ds(sid * (N_NODES // NS), N_NODES // NS)],
                        h_sh.at[pl.ds(sid * (N_NODES // NS), N_NODES // NS)])
        pltpu.sync_copy(src_hbm.at[pl.ds(base, PER_W)], idx_s)
        pltpu.sync_copy(dst_hbm.at[pl.ds(base, PER_W)], idx_d)
        plsc.subcore_barrier()

        @pl.loop(0, NBLK, step=_GK)
        def _(j0):
            s0 = j0 * E_BLK
            gathers = []
            for b in range(_GK):
                s = s0 + b * E_BLK
                gathers.append(pltpu.async_copy(
                    h_sh.at[idx_s.at[pl.ds(s, E_BLK)]], buf_s.at[b], gsem))
                gathers.append(pltpu.async_copy(
                    h_sh.at[idx_d.at[pl.ds(s, E_BLK)]], buf_d.at[b], gsem))
            writes = []
            for b in range(_GK):
                s = s0 + b * E_BLK
                gathers[2 * b].wait()
                writes.append(pltpu.async_copy(
                    buf_s.at[b], hs_hbm.at[pl.ds(base + s, E_BLK)], wsem))
                gathers[2 * b + 1].wait()
                writes.append(pltpu.async_copy(
                    buf_d.at[b], hd_hbm.at[pl.ds(base + s, E_BLK)], wsem))
            for w in writes:
                w.wait()

    return k(h_pk, src_idx, dst_idx)


def _sc_scatter(msg_e, msg_o, dste2d, dsto2d):
    @functools.partial(
        pl.kernel, mesh=plsc.VectorSubcoreMesh(**_MESH),
        out_type=jax.ShapeDtypeStruct((NC, N_PAD, D_STATE), jnp.float32),
        compiler_params=pltpu.CompilerParams(use_tc_tiling_on_sc=False),
        scratch_types=[
            pltpu.VMEM_SHARED((N_PAD, D_STATE), jnp.float32),
            pltpu.VMEM((NBLK_H, E_BLK), jnp.int32),
            pltpu.VMEM((NBLK_H, E_BLK), jnp.int32),
            pltpu.VMEM((_GK, E_BLK, D_STATE), jnp.float32),
            pltpu.VMEM((STRIPE, D_STATE), jnp.float32),
            pltpu.SemaphoreType.DMA,
        ])
    def k(me_hbm, mo_hbm, de_hbm, do_hbm, out_hbm, acc, idx_e, idx_o, mbuf,
          zbuf, lsem):
        cid = lax.axis_index("c")
        sid = lax.axis_index("s")
        wid = sid * NC + cid
        z = jnp.zeros((16,), jnp.float32)

        @pl.loop(0, STRIPE)
        def _(r):
            zbuf[r, pl.ds(0, 16)] = z
            zbuf[r, pl.ds(16, 16)] = z

        pltpu.sync_copy(zbuf, acc.at[pl.ds(sid * STRIPE, STRIPE)])

        pltpu.sync_copy(de_hbm.at[pl.ds(wid * NBLK_H, NBLK_H)], idx_e)
        pltpu.sync_copy(do_hbm.at[pl.ds(wid * NBLK_H, NBLK_H)], idx_o)
        plsc.subcore_barrier()

        def scat(src_hbm, idx):
            @pl.loop(0, NBLK_H, step=_GK)
            def _(j0):
                loads = []
                for b in range(_GK):
                    loads.append(pltpu.async_copy(
                        src_hbm.at[pl.ds(
                            wid * (NBLK_H * E_BLK) + (j0 + b) * E_BLK, E_BLK)],
                        mbuf.at[b], lsem))
                for b in range(_GK):
                    loads[b].wait()
                    pltpu.sync_copy(mbuf.at[b], acc.at[idx.at[j0 + b]],
                                    add=True)

        scat(me_hbm, idx_e)
        scat(mo_hbm, idx_o)

        plsc.subcore_barrier()
        pltpu.sync_copy(acc.at[pl.ds(sid * STRIPE, STRIPE)],
                        out_hbm.at[cid].at[pl.ds(sid * STRIPE, STRIPE)])

    return k(msg_e, msg_o, dste2d, dsto2d)


def _dot(a, b):
    return jnp.dot(a, b, preferred_element_type=jnp.float32)


def _unpack_bf16(x32):
    xi = lax.bitcast_convert_type(x32, jnp.int32)
    lo = lax.bitcast_convert_type(xi << 16, jnp.float32)
    hi = lax.bitcast_convert_type(
        xi & jnp.int32(-65536), jnp.float32)
    return lo.astype(jnp.bfloat16), hi.astype(jnp.bfloat16)


def _node_mlp_body(x_ref, w1_ref, b1_ref, w2_ref, b2_ref, o_ref):
    hid = jnp.maximum(_dot(x_ref[...], w1_ref[...]) + b1_ref[...], 0.0)
    o_ref[...] = _dot(hid, w2_ref[...]) + b2_ref[...]


def _node_mlp(x, w1, b1, w2, b2, d_out):
    return pl.pallas_call(
        _node_mlp_body,
        out_shape=jax.ShapeDtypeStruct((x.shape[0], d_out), jnp.float32),
    )(x, w1, b1, w2, b2)


def _node_mlp2_body(x_ref, w1_ref, b1_ref, w2_ref, b2_ref, o_ref, obf_ref):
    hid = jnp.maximum(_dot(x_ref[...], w1_ref[...]) + b1_ref[...], 0.0)
    o = _dot(hid, w2_ref[...]) + b2_ref[...]
    o_ref[...] = o
    obf_ref[...] = o.astype(jnp.bfloat16)


def _node_mlp2(x, w1, b1, w2, b2, d_out):
    return pl.pallas_call(
        _node_mlp2_body,
        out_shape=(jax.ShapeDtypeStruct((x.shape[0], d_out), jnp.float32),
                   jax.ShapeDtypeStruct((x.shape[0], d_out), jnp.bfloat16)),
    )(x, w1, b1, w2, b2)


def _edge_body(hs_ref, hd_ref, ef_ref, ws_lo_ref, ws_hi_ref, wd_lo_ref,
               wd_hi_ref, we_ref, b1_ref, w2m_ref, b2m_ref, w2a_ref, b2a_ref,
               oe_ref, oo_ref):
    bf = jnp.bfloat16
    s_lo, s_hi = _unpack_bf16(hs_ref[...])
    d_lo, d_hi = _unpack_bf16(hd_ref[...])
    u = (_dot(s_lo, ws_lo_ref[...]) + _dot(s_hi, ws_hi_ref[...])
         + _dot(d_lo, wd_lo_ref[...]) + _dot(d_hi, wd_hi_ref[...])
         + _dot(ef_ref[...], we_ref[...]) + b1_ref[...])
    u = jnp.maximum(u, 0.0).astype(bf)
    m = _dot(u, w2m_ref[...]) + b2m_ref[...]
    a = jax.nn.sigmoid(_dot(u, w2a_ref[...]) + b2a_ref[...])
    o = m * a
    oe_ref[...] = o[:, :128]
    oo_ref[...] = o[:, 128:]


_EB = 2048


def _edge_mlp(hs8, hd8, ef8, ws_lo, ws_hi, wd_lo, wd_hi, we, b1, w2m, b2m,
              w2a, b2a):
    full = lambda shape: pl.BlockSpec(shape, lambda i: (0, 0))
    ep8 = EDGE_PAD // 8
    return pl.pallas_call(
        _edge_body,
        grid=(ep8 // _EB,),
        in_specs=[
            pl.BlockSpec((_EB, 128), lambda i: (i, 0)),
            pl.BlockSpec((_EB, 128), lambda i: (i, 0)),
            pl.BlockSpec((_EB, 128), lambda i: (i, 0)),
            full((128, 512)), full((128, 512)), full((128, 512)),
            full((128, 512)), full((128, 512)), full((1, 512)),
            full((512, 256)), full((1, 256)), full((512, 256)),
            full((1, 256)),
        ],
        out_specs=(pl.BlockSpec((_EB, 128), lambda i: (i, 0)),
                   pl.BlockSpec((_EB, 128), lambda i: (i, 0))),
        out_shape=(jax.ShapeDtypeStruct((ep8, 128), jnp.float32),
                   jax.ShapeDtypeStruct((ep8, 128), jnp.float32)),
        compiler_params=pltpu.CompilerParams(
            dimension_semantics=("parallel",)),
    )(hs8, hd8, ef8, ws_lo, ws_hi, wd_lo, wd_hi, we, b1, w2m, b2m, w2a, b2a)


def _gru_body(p_ref, h_ref, wih_ref, bih_ref, whh_ref, bhh_ref, o_ref,
              obf_ref):
    ms = (p_ref[0] + p_ref[1])[:N_NODES]
    h = h_ref[...]
    gi = _dot(ms, wih_ref[...]) + bih_ref[...]
    gh = _dot(h, whh_ref[...]) + bhh_ref[...]
    r = jax.nn.sigmoid(gi[:, :D_STATE] + gh[:, :D_STATE])
    z = jax.nn.sigmoid(gi[:, D_STATE:2 * D_STATE] + gh[:, D_STATE:2 * D_STATE])
    n = jnp.tanh(gi[:, 2 * D_STATE:] + r * gh[:, 2 * D_STATE:])
    o = (1.0 - z) * n + z * h
    o_ref[...] = o
    obf_ref[...] = o.astype(jnp.bfloat16)


def _gru(part, h, wih, bih, whh, bhh):
    return pl.pallas_call(
        _gru_body,
        out_shape=(jax.ShapeDtypeStruct((N_NODES, D_STATE), jnp.float32),
                   jax.ShapeDtypeStruct((N_NODES, D_STATE), jnp.bfloat16)),
    )(part, h, wih, bih, whh, bhh)


def _res_body(h_ref, old_ref, w_ref, b_ref, o_ref, orelu_ref, orelubf_ref):
    x = (_dot(h_ref[...], w_ref[:D_STATE]) + _dot(old_ref[...], w_ref[D_STATE:])
         + b_ref[...])
    o_ref[...] = x
    xr = jnp.maximum(x, 0.0)
    orelu_ref[...] = xr
    orelubf_ref[...] = xr.astype(jnp.bfloat16)


def _res(h, old, w, b):
    return pl.pallas_call(
        _res_body,
        out_shape=(jax.ShapeDtypeStruct((N_NODES, D_STATE), jnp.float32),
                   jax.ShapeDtypeStruct((N_NODES, D_STATE), jnp.float32),
                   jax.ShapeDtypeStruct((N_NODES, D_STATE), jnp.bfloat16)),
    )(h, old, w, b)


def _pack_pairs(h_bf):
    return lax.bitcast_convert_type(
        h_bf.reshape(h_bf.shape[0], D_PK, 2), jnp.float32)


def kernel(nodes_feature, edges, edges_feature, params):
    p = params
    n_edges = edges.shape[0]
    npad = EDGE_PAD - n_edges
    src = edges[:, 0].astype(jnp.int32)
    dst = edges[:, 1].astype(jnp.int32)
    il4 = lambda a, b: jnp.stack(
        [a.reshape(-1, 4), b.reshape(-1, 4)], axis=1).reshape(-1)
    srcp = jnp.pad(src, (0, npad))
    dstp = jnp.pad(dst, (0, npad))
    srcp = il4(srcp[:EPH], srcp[EPH:])
    dstp = il4(dstp[:EPH], dstp[EPH:])
    dst_sc = jnp.pad(dst, (0, npad), constant_values=N_NODES)
    dste2d = dst_sc[:EPH].reshape(EPH // E_BLK, E_BLK)
    dsto2d = dst_sc[EPH:].reshape(EPH // E_BLK, E_BLK)
    efb = edges_feature.astype(jnp.bfloat16)
    ef_a = efb[:EPH].reshape(EPH // 4, 64)
    ef_b = jnp.pad(efb[EPH:].reshape((n_edges - EPH) // 4, 64),
                   ((0, npad // 4), (0, 0)))
    ef8 = jnp.concatenate([ef_a, ef_b], axis=1)

    h, h_bf = _node_mlp2(nodes_feature, p['in_W1'].T, p['in_b1'][None],
                         p['in_W2'].T, p['in_b2'][None], D_STATE)

    h_relu = None
    h_relu_bf = None
    for i in range(2):
        old = h
        if i > 0:
            h = h_relu
            h_bf = h_relu_bf
        mW1, aW1 = p['msg_W1_%d' % i], p['att_W1_%d' % i]
        eye8 = jnp.eye(8, dtype=jnp.float32)
        blk8 = lambda w: jnp.kron(eye8, w)
        w1s = blk8(jnp.concatenate([mW1[:, :32], aW1[:, :32]], 0).T)
        w1d = blk8(jnp.concatenate([mW1[:, 32:64], aW1[:, 32:64]], 0).T)
        bfc = lambda w: w.astype(jnp.bfloat16)
        ws_lo, ws_hi = bfc(w1s[0::2]), bfc(w1s[1::2])
        wd_lo, wd_hi = bfc(w1d[0::2]), bfc(w1d[1::2])
        we = bfc(blk8(jnp.concatenate([mW1[:, 64:], aW1[:, 64:]], 0).T))
        b1 = jnp.tile(
            jnp.concatenate([p['msg_b1_%d' % i], p['att_b1_%d' % i]]), 8)[None]
        zz = jnp.zeros((D_STATE, D_STATE), jnp.float32)
        w2m = bfc(blk8(jnp.concatenate([p['msg_W2_%d' % i].T, zz], 0)))
        w2a = bfc(blk8(jnp.concatenate([zz, p['att_W2_%d' % i].T], 0)))
        b2m = jnp.tile(p['msg_b2_%d' % i], 8)[None]
        b2a = jnp.tile(p['att_b2_%d' % i], 8)[None]
        wih, bih = p['gru_Wih_%d' % i].T, p['gru_bih_%d' % i][None]
        whh, bhh = p['gru_Whh_%d' % i].T, p['gru_bhh_%d' % i][None]
        for _ in range(2):
            hpk = _pack_pairs(h_bf)
            hs, hd = _sc_gather(hpk, srcp, dstp)
            hs8 = hs.reshape(EDGE_PAD // 8, 128)
            hd8 = hd.reshape(EDGE_PAD // 8, 128)
            msg_e, msg_o = _edge_mlp(hs8, hd8, ef8, ws_lo, ws_hi, wd_lo,
                                     wd_hi, we, b1, w2m, b2m, w2a, b2a)
            part = _sc_scatter(msg_e.reshape(EPH, D_STATE),
                               msg_o.reshape(EPH, D_STATE), dste2d, dsto2d)
            h, h_bf = _gru(part, h, wih, bih, whh, bhh)
        h, h_relu, h_relu_bf = _res(h, old, p['res_W_%d' % i].T,
                                    p['res_b_%d' % i][None])

    return _node_mlp(h, p['ro_W1'].T, p['ro_b1'][None],
                     p['ro_W2'].T, p['ro_b2'][None], 64)


# speedup vs baseline: 1.4648x; 1.4648x over previous
import functools

import jax
import jax.numpy as jnp
from jax import lax
from jax.experimental import pallas as pl
from jax.experimental.pallas import tpu as pltpu
from jax.experimental.pallas import tpu_sc as plsc

N_NODES = 10000
D_STATE = 32
NC, NS = 2, 16
NW = NC * NS
E_BLK = 128
EDGE_PAD = 327680
PER_W = EDGE_PAD // NW
NBLK = PER_W // E_BLK
N_PAD = 10016
STRIPE = N_PAD // NS

_MESH = dict(core_axis_name="c", subcore_axis_name="s")
_GK = 4


def _sc_gather(h_tbl, src_idx, dst_idx):
    out = (jax.ShapeDtypeStruct((EDGE_PAD, D_STATE), jnp.float32),
           jax.ShapeDtypeStruct((EDGE_PAD, D_STATE), jnp.float32))

    @functools.partial(
        pl.kernel, mesh=plsc.VectorSubcoreMesh(**_MESH), out_type=out,
        compiler_params=pltpu.CompilerParams(use_tc_tiling_on_sc=False),
        scratch_types=[
            pltpu.VMEM_SHARED((N_NODES, D_STATE), jnp.float32),
            pltpu.VMEM((PER_W,), jnp.int32),
            pltpu.VMEM((PER_W,), jnp.int32),
            pltpu.VMEM((_GK, E_BLK, D_STATE), jnp.float32),
            pltpu.VMEM((_GK, E_BLK, D_STATE), jnp.float32),
            pltpu.SemaphoreType.DMA,
            pltpu.SemaphoreType.DMA,
        ])
    def k(h_hbm, src_hbm, dst_hbm, hs_hbm, hd_hbm, h_sh, idx_s, idx_d, buf_s,
          buf_d, gsem, wsem):
        sid = lax.axis_index("s")
        wid = sid * NC + lax.axis_index("c")
        base = wid * PER_W
        pltpu.sync_copy(h_hbm.at[pl.ds(sid * (N_NODES // NS), N_NODES // NS)],
                        h_sh.at[pl.ds(sid * (N_NODES // NS), N_NODES // NS)])
        pltpu.sync_copy(src_hbm.at[pl.ds(base, PER_W)], idx_s)
        pltpu.sync_copy(dst_hbm.at[pl.ds(base, PER_W)], idx_d)
        plsc.subcore_barrier()

        @pl.loop(0, NBLK, step=_GK)
        def _(j0):
            s0 = j0 * E_BLK
            gathers = []
            for b in range(_GK):
                s = s0 + b * E_BLK
                gathers.append(pltpu.async_copy(
                    h_sh.at[idx_s.at[pl.ds(s, E_BLK)]], buf_s.at[b], gsem))
                gathers.append(pltpu.async_copy(
                    h_sh.at[idx_d.at[pl.ds(s, E_BLK)]], buf_d.at[b], gsem))
            writes = []
            for b in range(_GK):
                s = s0 + b * E_BLK
                gathers[2 * b].wait()
                writes.append(pltpu.async_copy(
                    buf_s.at[b], hs_hbm.at[pl.ds(base + s, E_BLK)], wsem))
                gathers[2 * b + 1].wait()
                writes.append(pltpu.async_copy(
                    buf_d.at[b], hd_hbm.at[pl.ds(base + s, E_BLK)], wsem))
            for w in writes:
                w.wait()

    return k(h_tbl, src_idx, dst_idx)


def _sc_scatter(msg, dst2d):
    @functools.partial(
        pl.kernel, mesh=plsc.VectorSubcoreMesh(**_MESH),
        out_type=jax.ShapeDtypeStruct((NC, N_PAD, D_STATE), jnp.float32),
        compiler_params=pltpu.CompilerParams(use_tc_tiling_on_sc=False),
        scratch_types=[
            pltpu.VMEM_SHARED((N_PAD, D_STATE), jnp.float32),
            pltpu.VMEM((NBLK, E_BLK), jnp.int32),
            pltpu.VMEM((_GK, E_BLK, D_STATE), jnp.float32),
            pltpu.VMEM((STRIPE, D_STATE), jnp.float32),
            pltpu.SemaphoreType.DMA,
        ])
    def k(msg_hbm, dst_hbm, out_hbm, acc, idx, mbuf, zbuf, lsem):
        cid = lax.axis_index("c")
        sid = lax.axis_index("s")
        wid = sid * NC + cid
        z = jnp.zeros((16,), jnp.float32)

        @pl.loop(0, STRIPE)
        def _(r):
            zbuf[r, pl.ds(0, 16)] = z
            zbuf[r, pl.ds(16, 16)] = z

        pltpu.sync_copy(zbuf, acc.at[pl.ds(sid * STRIPE, STRIPE)])
        plsc.subcore_barrier()

        pltpu.sync_copy(dst_hbm.at[pl.ds(wid * NBLK, NBLK)], idx)

        @pl.loop(0, NBLK, step=_GK)
        def _(j0):
            loads = []
            for b in range(_GK):
                loads.append(pltpu.async_copy(
                    msg_hbm.at[pl.ds(wid * PER_W + (j0 + b) * E_BLK, E_BLK)],
                    mbuf.at[b], lsem))
            for b in range(_GK):
                loads[b].wait()
                pltpu.sync_copy(mbuf.at[b], acc.at[idx.at[j0 + b]], add=True)

        plsc.subcore_barrier()
        pltpu.sync_copy(acc.at[pl.ds(sid * STRIPE, STRIPE)],
                        out_hbm.at[cid].at[pl.ds(sid * STRIPE, STRIPE)])

    return k(msg, dst2d)


def _dot(a, b):
    return jnp.dot(a, b, preferred_element_type=jnp.float32)


def _node_mlp_body(x_ref, w1_ref, b1_ref, w2_ref, b2_ref, o_ref):
    hid = jnp.maximum(_dot(x_ref[...], w1_ref[...]) + b1_ref[...], 0.0)
    o_ref[...] = _dot(hid, w2_ref[...]) + b2_ref[...]


def _node_mlp(x, w1, b1, w2, b2, d_out):
    return pl.pallas_call(
        _node_mlp_body,
        out_shape=jax.ShapeDtypeStruct((x.shape[0], d_out), jnp.float32),
    )(x, w1, b1, w2, b2)


def _edge_body(hs_ref, hd_ref, ef_ref, w1s_ref, w1d_ref, w1e_ref, b1_ref,
               w2m_ref, b2m_ref, w2a_ref, b2a_ref, o_ref):
    bf = jnp.bfloat16
    u = (_dot(hs_ref[...].astype(bf), w1s_ref[...])
         + _dot(hd_ref[...].astype(bf), w1d_ref[...])
         + _dot(ef_ref[...].astype(bf), w1e_ref[...]) + b1_ref[...])
    u = jnp.maximum(u, 0.0).astype(bf)
    m = _dot(u, w2m_ref[...]) + b2m_ref[...]
    a = jax.nn.sigmoid(_dot(u, w2a_ref[...]) + b2a_ref[...])
    o_ref[...] = m * a


_EB = 2048


def _edge_mlp(hs, hd, ef4, w1s, w1d, w1e, b1, w2m, b2m, w2a, b2a):
    full = lambda shape: pl.BlockSpec(shape, lambda i: (0, 0))
    ep4 = EDGE_PAD // 4
    return pl.pallas_call(
        _edge_body,
        grid=(ep4 // _EB,),
        in_specs=[
            pl.BlockSpec((_EB, 128), lambda i: (i, 0)),
            pl.BlockSpec((_EB, 128), lambda i: (i, 0)),
            pl.BlockSpec((_EB, 64), lambda i: (i, 0)),
            full((128, 256)), full((128, 256)), full((64, 256)),
            full((1, 256)), full((256, 128)), full((1, 128)),
            full((256, 128)), full((1, 128)),
        ],
        out_specs=pl.BlockSpec((_EB, 128), lambda i: (i, 0)),
        out_shape=jax.ShapeDtypeStruct((ep4, 128), jnp.float32),
        compiler_params=pltpu.CompilerParams(
            dimension_semantics=("parallel",)),
    )(hs, hd, ef4, w1s, w1d, w1e, b1, w2m, b2m, w2a, b2a)


def _gru_body(p_ref, h_ref, wih_ref, bih_ref, whh_ref, bhh_ref, o_ref):
    ms = (p_ref[0] + p_ref[1])[:N_NODES]
    h = h_ref[...]
    gi = _dot(ms, wih_ref[...]) + bih_ref[...]
    gh = _dot(h, whh_ref[...]) + bhh_ref[...]
    r = jax.nn.sigmoid(gi[:, :D_STATE] + gh[:, :D_STATE])
    z = jax.nn.sigmoid(gi[:, D_STATE:2 * D_STATE] + gh[:, D_STATE:2 * D_STATE])
    n = jnp.tanh(gi[:, 2 * D_STATE:] + r * gh[:, 2 * D_STATE:])
    o_ref[...] = (1.0 - z) * n + z * h


def _gru(part, h, wih, bih, whh, bhh):
    return pl.pallas_call(
        _gru_body,
        out_shape=jax.ShapeDtypeStruct((N_NODES, D_STATE), jnp.float32),
    )(part, h, wih, bih, whh, bhh)


def _res_body(h_ref, old_ref, w_ref, b_ref, o_ref, orelu_ref):
    x = (_dot(h_ref[...], w_ref[:D_STATE]) + _dot(old_ref[...], w_ref[D_STATE:])
         + b_ref[...])
    o_ref[...] = x
    orelu_ref[...] = jnp.maximum(x, 0.0)


def _res(h, old, w, b):
    return pl.pallas_call(
        _res_body,
        out_shape=(jax.ShapeDtypeStruct((N_NODES, D_STATE), jnp.float32),
                   jax.ShapeDtypeStruct((N_NODES, D_STATE), jnp.float32)),
    )(h, old, w, b)


def kernel(nodes_feature, edges, edges_feature, params):
    p = params
    n_edges = edges.shape[0]
    npad = EDGE_PAD - n_edges
    src = edges[:, 0].astype(jnp.int32)
    dst = edges[:, 1].astype(jnp.int32)
    srcp = jnp.pad(src, (0, npad))
    dstp = jnp.pad(dst, (0, npad))
    dst_sc = jnp.pad(dst, (0, npad), constant_values=N_NODES)
    dst2d = dst_sc.reshape(EDGE_PAD // E_BLK, E_BLK)
    ef4 = jnp.pad(edges_feature.reshape(n_edges // 4, 64),
                  ((0, npad // 4), (0, 0)))

    h = _node_mlp(nodes_feature, p['in_W1'].T, p['in_b1'][None],
                  p['in_W2'].T, p['in_b2'][None], D_STATE)

    h_relu = None
    for i in range(2):
        old = h
        if i > 0:
            h = h_relu
        mW1, aW1 = p['msg_W1_%d' % i], p['att_W1_%d' % i]
        eye4 = jnp.eye(4, dtype=jnp.float32)
        blk4 = lambda w: jnp.kron(eye4, w).astype(jnp.bfloat16)
        w1s = blk4(jnp.concatenate([mW1[:, :32], aW1[:, :32]], 0).T)
        w1d = blk4(jnp.concatenate([mW1[:, 32:64], aW1[:, 32:64]], 0).T)
        w1e = blk4(jnp.concatenate([mW1[:, 64:], aW1[:, 64:]], 0).T)
        b1 = jnp.tile(
            jnp.concatenate([p['msg_b1_%d' % i], p['att_b1_%d' % i]]), 4)[None]
        zz = jnp.zeros((D_STATE, D_STATE), jnp.float32)
        w2m = blk4(jnp.concatenate([p['msg_W2_%d' % i].T, zz], 0))
        w2a = blk4(jnp.concatenate([zz, p['att_W2_%d' % i].T], 0))
        b2m = jnp.tile(p['msg_b2_%d' % i], 4)[None]
        b2a = jnp.tile(p['att_b2_%d' % i], 4)[None]
        wih, bih = p['gru_Wih_%d' % i].T, p['gru_bih_%d' % i][None]
        whh, bhh = p['gru_Whh_%d' % i].T, p['gru_bhh_%d' % i][None]
        for _ in range(2):
            hs, hd = _sc_gather(h, srcp, dstp)
            hs4 = hs.reshape(EDGE_PAD // 4, 128)
            hd4 = hd.reshape(EDGE_PAD // 4, 128)
            msg4 = _edge_mlp(hs4, hd4, ef4, w1s, w1d, w1e, b1, w2m, b2m,
                             w2a, b2a)
            part = _sc_scatter(msg4.reshape(EDGE_PAD, D_STATE), dst2d)
            h = _gru(part, h, wih, bih, whh, bhh)
        h, h_relu = _res(h, old, p['res_W_%d' % i].T, p['res_b_%d' % i][None])

    return _node_mlp(h, p['ro_W1'].T, p['ro_b1'][None],
                     p['ro_W2'].T, p['ro_b2'][None], 64)
